# baseline (speedup 1.0000x reference)
import functools

import jax
import jax.numpy as jnp
from jax import lax
from jax.experimental import pallas as pl
from jax.experimental.pallas import tpu as pltpu
from jax.experimental.pallas import tpu_sc as plsc

KNN = 16
TW = 256
NEG_INF = float("-inf")

SC_NC = 2
SC_NS = 16
SC_NW = SC_NC * SC_NS


def _prep_body(fea_ref, fea_s_ref, xyzT_ref, xyz_sT_ref, wf1_ref, wf2_ref,
               wge_ref, bf_ref, bg_ref, t_ref, ap_ref):
    fea = fea_ref[0]
    fea_s = fea_s_ref[0]
    xq = xyzT_ref[0]
    xs = xyz_sT_ref[0]
    dn = (((0,), (0,)), ((), ()))
    a = lax.dot_general(fea, wf1_ref[...], dn,
                        preferred_element_type=jnp.float32)
    ap_ref[:, 0:128] = a + bf_ref[...]
    p = jnp.dot(xq, wge_ref[...], preferred_element_type=jnp.float32)
    ap_ref[:, 128:256] = p + bg_ref[...]
    bm = lax.dot_general(fea_s, wf2_ref[...], dn,
                         preferred_element_type=jnp.float32)
    t_ref[:, 0:128] = bm
    t_ref[:, 128:132] = xs
    t_ref[:, 132:256] = jnp.zeros_like(t_ref[:, 132:256])


def _prep(fea, fea_s, xyzT, xyz_sT, wf1, wf2, wge, bf2, bg2):
    b, c, n = fea.shape
    pb = 512
    nb = n // pb
    grid = (b, nb)
    out_c = wf1.shape[1]
    return pl.pallas_call(
        _prep_body,
        grid=grid,
        in_specs=[
            pl.BlockSpec((1, c, pb), lambda i, j: (i, 0, j)),
            pl.BlockSpec((1, c, pb), lambda i, j: (i, 0, j)),
            pl.BlockSpec((1, pb, 4), lambda i, j: (i, j, 0)),
            pl.BlockSpec((1, pb, 4), lambda i, j: (i, j, 0)),
            pl.BlockSpec((c, out_c), lambda i, j: (0, 0)),
            pl.BlockSpec((c, out_c), lambda i, j: (0, 0)),
            pl.BlockSpec((4, out_c), lambda i, j: (0, 0)),
            pl.BlockSpec((1, out_c), lambda i, j: (0, 0)),
            pl.BlockSpec((1, out_c), lambda i, j: (0, 0)),
        ],
        out_specs=[
            pl.BlockSpec((pb, TW), lambda i, j: (i * nb + j, 0)),
            pl.BlockSpec((pb, 2 * out_c), lambda i, j: (i * nb + j, 0)),
        ],
        out_shape=[
            jax.ShapeDtypeStruct((b * n, TW), jnp.float32),
            jax.ShapeDtypeStruct((b * n, 2 * out_c), jnp.float32),
        ],
        compiler_params=pltpu.CompilerParams(
            dimension_semantics=("parallel", "parallel")),
    )(fea, fea_s, xyzT, xyz_sT, wf1, wf2, wge, bf2, bg2)


def _knn_body(xyzT_ref, xyz_s_ref, idx_ref, *, n, qb, k):
    bi = pl.program_id(0)
    xq = xyzT_ref[0]
    xs = xyz_s_ref[0]
    xqb = xq.astype(jnp.bfloat16).astype(jnp.float32)
    xsb = xs.astype(jnp.bfloat16).astype(jnp.float32)
    e = (xqb[:, 0:1] * xsb[0:1, :]
         + xqb[:, 1:2] * xsb[1:2, :]) + xqb[:, 2:3] * xsb[2:3, :]
    inner = -2.0 * e
    a2 = (xq[:, 0:1] * xq[:, 0:1] + xq[:, 1:2] * xq[:, 1:2]) \
        + xq[:, 2:3] * xq[:, 2:3]
    b2 = (xs[0:1, :] * xs[0:1, :] + xs[1:2, :] * xs[1:2, :]) \
        + xs[2:3, :] * xs[2:3, :]
    dis = (-a2 - inner) - b2
    iota = lax.broadcasted_iota(jnp.int32, (qb, n), 1)
    cols = []
    d = dis
    for _ in range(k):
        m = jnp.max(d, axis=1, keepdims=True)
        sel = jnp.min(jnp.where(d == m, iota, jnp.int32(n)),
                      axis=1, keepdims=True)
        cols.append(sel)
        d = jnp.where(iota == sel, NEG_INF, d)
    idx_ref[0] = jnp.concatenate(cols, axis=1) + bi * n


def _knn(xyzT, xyz_s):
    b, n, _ = xyzT.shape
    qb = 128
    grid = (b, n // qb)
    body = functools.partial(_knn_body, n=n, qb=qb, k=KNN)
    return pl.pallas_call(
        body,
        grid=grid,
        in_specs=[
            pl.BlockSpec((1, qb, 4), lambda i, j: (i, j, 0)),
            pl.BlockSpec((1, 3, n), lambda i, j: (i, 0, 0)),
        ],
        out_specs=pl.BlockSpec((1, qb, KNN), lambda i, j: (i, j, 0)),
        out_shape=jax.ShapeDtypeStruct((b, n, KNN), jnp.int32),
        compiler_params=pltpu.CompilerParams(
            dimension_semantics=("parallel", "parallel")),
    )(xyzT, xyz_s)


def _gather_rows(table, idx):
    m = idx.shape[0]
    tw = table.shape[1]
    per_w = m // SC_NW
    ch = 128
    n_ch = per_w // ch
    mesh = plsc.VectorSubcoreMesh(core_axis_name="c", subcore_axis_name="s")

    @functools.partial(
        pl.kernel,
        out_type=jax.ShapeDtypeStruct((m, tw), jnp.float32),
        mesh=mesh,
        scratch_types=[
            pltpu.VMEM((ch,), jnp.int32),
            pltpu.VMEM((ch, tw), jnp.float32),
            pltpu.SemaphoreType.DMA,
        ],
    )
    def gather_k(t_hbm, idx_hbm, out_hbm, idx_v, rows_v, sem):
        wid = lax.axis_index("s") * SC_NC + lax.axis_index("c")
        base = wid * per_w

        def body(i, carry):
            off = base + i * ch
            pltpu.sync_copy(idx_hbm.at[pl.ds(off, ch)], idx_v)
            pltpu.async_copy(t_hbm.at[idx_v], rows_v, sem).wait()
            pltpu.sync_copy(rows_v, out_hbm.at[pl.ds(off, ch)])
            return carry

        lax.fori_loop(0, n_ch, body, 0)

    return gather_k(table, idx)


def _final_body(g_ref, ap_ref, xyzT_ref, w0_ref, wgs_ref, out_ref,
                *, fb, k, tw):
    flat = g_ref[...]
    q = jnp.dot(flat[:, 128:132], wgs_ref[...],
                preferred_element_type=jnp.float32)
    r = jnp.reshape(flat, (fb, k, tw))
    gf = r[:, :, 0:128]
    gx = r[:, :, 128:131]
    ap = ap_ref[...]
    a = ap[:, None, 0:128]
    p = ap[:, None, 128:256]
    xq = xyzT_ref[0][:, None, 0:3]
    diff = xq - gx
    d = jnp.sqrt(jnp.sum(diff * diff, axis=2, keepdims=True))
    f = jnp.maximum(a + gf, 0.0)
    g = jnp.maximum(p + jnp.reshape(q, (fb, k, 128))
                    + w0_ref[...][None] * d, 0.0)
    out_ref[0] = jnp.max(f * g, axis=1)


def _final(g, ap, xyzT, w0, wgs):
    b, n, _ = xyzT.shape
    out_c = w0.shape[1]
    fb = 256
    nb = n // fb
    body = functools.partial(_final_body, fb=fb, k=KNN, tw=TW)
    return pl.pallas_call(
        body,
        grid=(b, nb),
        in_specs=[
            pl.BlockSpec((fb * KNN, TW), lambda i, j: (i * nb + j, 0)),
            pl.BlockSpec((fb, 2 * out_c), lambda i, j: (i * nb + j, 0)),
            pl.BlockSpec((1, fb, 4), lambda i, j: (i, j, 0)),
            pl.BlockSpec((1, out_c), lambda i, j: (0, 0)),
            pl.BlockSpec((4, out_c), lambda i, j: (0, 0)),
        ],
        out_specs=pl.BlockSpec((1, fb, out_c), lambda i, j: (i, j, 0)),
        out_shape=jax.ShapeDtypeStruct((b, n, out_c), jnp.float32),
        compiler_params=pltpu.CompilerParams(
            dimension_semantics=("parallel", "parallel")),
    )(g, ap, xyzT, w0, wgs)


def kernel(xyz, xyz_s, fea, fea_s, Wf, bf, Wg, bg):
    b, c, n = fea.shape
    out_c = Wf.shape[0]
    pad = jnp.zeros((b, n, 1), jnp.float32)
    xyzT = jnp.concatenate([jnp.swapaxes(xyz, 1, 2), pad], axis=2)
    xyz_sT = jnp.concatenate([jnp.swapaxes(xyz_s, 1, 2), pad], axis=2)
    wf1 = Wf[:, :c].T
    wf2 = Wf[:, c:].T
    wpad = jnp.zeros((1, out_c), jnp.float32)
    wge = jnp.concatenate([(Wg[:, 1:4] + Wg[:, 7:10]).T, wpad], axis=0)
    wgs = jnp.concatenate([(Wg[:, 4:7] - Wg[:, 7:10]).T, wpad], axis=0)
    w0 = Wg[:, 0:1].T
    bf2 = bf.reshape(1, out_c)
    bg2 = bg.reshape(1, out_c)

    table, ap = _prep(fea, fea_s, xyzT, xyz_sT, wf1, wf2, wge, bf2, bg2)
    idx = _knn(xyzT, xyz_s)
    gathered = _gather_rows(table, idx.reshape(-1))
    out = _final(gathered, ap, xyzT, w0, wgs)
    return jnp.swapaxes(out, 1, 2)


# speedup vs baseline: 1.5359x; 1.5359x over previous
import functools

import jax
import jax.numpy as jnp
from jax import lax
from jax.experimental import pallas as pl
from jax.experimental.pallas import tpu as pltpu
from jax.experimental.pallas import tpu_sc as plsc

KNN = 16
TW = 256
NEG_INF = float("-inf")

SC_NC = 2
SC_NS = 16
SC_NW = SC_NC * SC_NS


def _prep_body(fea_ref, fea_s_ref, xyzT_ref, xyz_sT_ref, wf1_ref, wf2_ref,
               wge_ref, bf_ref, bg_ref, t_ref, ap_ref):
    fea = fea_ref[0]
    fea_s = fea_s_ref[0]
    xq = xyzT_ref[0]
    xs = xyz_sT_ref[0]
    dn = (((0,), (0,)), ((), ()))
    a = lax.dot_general(fea, wf1_ref[...], dn,
                        preferred_element_type=jnp.float32)
    ap_ref[:, 0:128] = a + bf_ref[...]
    p = jnp.dot(xq, wge_ref[...], preferred_element_type=jnp.float32)
    ap_ref[:, 128:256] = p + bg_ref[...]
    bm = lax.dot_general(fea_s, wf2_ref[...], dn,
                         preferred_element_type=jnp.float32)
    t_ref[:, 0:128] = bm
    t_ref[:, 128:132] = xs
    t_ref[:, 132:256] = jnp.zeros_like(t_ref[:, 132:256])


def _prep(fea, fea_s, xyzT, xyz_sT, wf1, wf2, wge, bf2, bg2):
    b, c, n = fea.shape
    pb = 512
    nb = n // pb
    grid = (b, nb)
    out_c = wf1.shape[1]
    return pl.pallas_call(
        _prep_body,
        grid=grid,
        in_specs=[
            pl.BlockSpec((1, c, pb), lambda i, j: (i, 0, j)),
            pl.BlockSpec((1, c, pb), lambda i, j: (i, 0, j)),
            pl.BlockSpec((1, pb, 4), lambda i, j: (i, j, 0)),
            pl.BlockSpec((1, pb, 4), lambda i, j: (i, j, 0)),
            pl.BlockSpec((c, out_c), lambda i, j: (0, 0)),
            pl.BlockSpec((c, out_c), lambda i, j: (0, 0)),
            pl.BlockSpec((4, out_c), lambda i, j: (0, 0)),
            pl.BlockSpec((1, out_c), lambda i, j: (0, 0)),
            pl.BlockSpec((1, out_c), lambda i, j: (0, 0)),
        ],
        out_specs=[
            pl.BlockSpec((pb, TW), lambda i, j: (i * nb + j, 0)),
            pl.BlockSpec((pb, 2 * out_c), lambda i, j: (i * nb + j, 0)),
        ],
        out_shape=[
            jax.ShapeDtypeStruct((b * n, TW), jnp.float32),
            jax.ShapeDtypeStruct((b * n, 2 * out_c), jnp.float32),
        ],
        compiler_params=pltpu.CompilerParams(
            dimension_semantics=("parallel", "parallel")),
    )(fea, fea_s, xyzT, xyz_sT, wf1, wf2, wge, bf2, bg2)


def _ce(va, ia, vb, ib):
    c = vb > va
    return (jnp.maximum(va, vb), jnp.where(c, ib, ia),
            jnp.minimum(va, vb), jnp.where(c, ia, ib))


def _sort16_desc(v, ix):
    k = 2
    while k <= 16:
        j = k // 2
        while j >= 1:
            for i in range(16):
                l = i ^ j
                if l > i:
                    if (i & k) == 0:
                        v[i], ix[i], v[l], ix[l] = _ce(v[i], ix[i],
                                                       v[l], ix[l])
                    else:
                        v[l], ix[l], v[i], ix[i] = _ce(v[l], ix[l],
                                                       v[i], ix[i])
            j //= 2
        k *= 2
    return v, ix


def _merge16_desc(av, ai, bv, bi):
    mv, mi = [], []
    for i in range(16):
        c = bv[15 - i] > av[i]
        mv.append(jnp.maximum(av[i], bv[15 - i]))
        mi.append(jnp.where(c, bi[15 - i], ai[i]))
    for j in (8, 4, 2, 1):
        for i in range(16):
            l = i ^ j
            if l > i:
                mv[i], mi[i], mv[l], mi[l] = _ce(mv[i], mi[i], mv[l], mi[l])
    return mv, mi


def _knn_body(xyzT_ref, xyz_s_ref, idx_ref, *, n, qb, k):
    bi = pl.program_id(0)
    xq = xyzT_ref[0]
    xs = xyz_s_ref[0]
    xqb = xq.astype(jnp.bfloat16).astype(jnp.float32)
    xsb = xs.astype(jnp.bfloat16).astype(jnp.float32)
    e = (xqb[:, 0:1] * xsb[0:1, :]
         + xqb[:, 1:2] * xsb[1:2, :]) + xqb[:, 2:3] * xsb[2:3, :]
    inner = -2.0 * e
    a2 = (xq[:, 0:1] * xq[:, 0:1] + xq[:, 1:2] * xq[:, 1:2]) \
        + xq[:, 2:3] * xq[:, 2:3]
    b2 = (xs[0:1, :] * xs[0:1, :] + xs[1:2, :] * xs[1:2, :]) \
        + xs[2:3, :] * xs[2:3, :]
    dis = (-a2 - inner) - b2

    nw = n // 128
    lane = lax.broadcasted_iota(jnp.int32, (qb, 128), 1)
    ninf = jnp.full((qb, 128), NEG_INF, jnp.float32)
    groups = []
    for g0 in range(0, nw, 16):
        gv, gi = [], []
        for i in range(16):
            if g0 + i < nw:
                gv.append(dis[:, (g0 + i) * 128:(g0 + i + 1) * 128])
                gi.append(lane + (g0 + i) * 128)
            else:
                gv.append(ninf)
                gi.append(lane)
        groups.append(_sort16_desc(gv, gi))
    while len(groups) > 1:
        nxt = []
        for a in range(0, len(groups), 2):
            nxt.append(_merge16_desc(groups[a][0], groups[a][1],
                                     groups[a + 1][0], groups[a + 1][1]))
        groups = nxt
    wv, wi = groups[0]

    cols = []
    for _ in range(k):
        m = jnp.max(wv[0], axis=1, keepdims=True)
        lsel = jnp.min(jnp.where(wv[0] == m, lane, jnp.int32(128)),
                       axis=1, keepdims=True)
        hit = lane == lsel
        cols.append(jnp.sum(jnp.where(hit, wi[0], 0), axis=1, keepdims=True))
        for t in range(15):
            wv[t] = jnp.where(hit, wv[t + 1], wv[t])
            wi[t] = jnp.where(hit, wi[t + 1], wi[t])
        wv[15] = jnp.where(hit, NEG_INF, wv[15])
    idx_ref[0] = jnp.concatenate(cols, axis=1) + bi * n


def _knn(xyzT, xyz_s):
    b, n, _ = xyzT.shape
    qb = 128
    grid = (b, n // qb)
    body = functools.partial(_knn_body, n=n, qb=qb, k=KNN)
    return pl.pallas_call(
        body,
        grid=grid,
        in_specs=[
            pl.BlockSpec((1, qb, 4), lambda i, j: (i, j, 0)),
            pl.BlockSpec((1, 3, n), lambda i, j: (i, 0, 0)),
        ],
        out_specs=pl.BlockSpec((1, qb, KNN), lambda i, j: (i, j, 0)),
        out_shape=jax.ShapeDtypeStruct((b, n, KNN), jnp.int32),
        compiler_params=pltpu.CompilerParams(
            dimension_semantics=("parallel", "parallel")),
    )(xyzT, xyz_s)


def _gather_rows(table, idx):
    m = idx.shape[0]
    tw = table.shape[1]
    per_w = m // SC_NW
    ch = 128
    n_ch = per_w // ch
    mesh = plsc.VectorSubcoreMesh(core_axis_name="c", subcore_axis_name="s")

    @functools.partial(
        pl.kernel,
        out_type=jax.ShapeDtypeStruct((m, tw), jnp.float32),
        mesh=mesh,
        scratch_types=[
            pltpu.VMEM((ch,), jnp.int32),
            pltpu.VMEM((ch, tw), jnp.float32),
            pltpu.SemaphoreType.DMA,
        ],
    )
    def gather_k(t_hbm, idx_hbm, out_hbm, idx_v, rows_v, sem):
        wid = lax.axis_index("s") * SC_NC + lax.axis_index("c")
        base = wid * per_w

        def body(i, carry):
            off = base + i * ch
            pltpu.sync_copy(idx_hbm.at[pl.ds(off, ch)], idx_v)
            pltpu.async_copy(t_hbm.at[idx_v], rows_v, sem).wait()
            pltpu.sync_copy(rows_v, out_hbm.at[pl.ds(off, ch)])
            return carry

        lax.fori_loop(0, n_ch, body, 0)

    return gather_k(table, idx)


def _final_body(g_ref, ap_ref, xyzT_ref, w0_ref, wgs_ref, out_ref,
                *, fb, k, tw):
    flat = g_ref[...]
    q = jnp.dot(flat[:, 128:132], wgs_ref[...],
                preferred_element_type=jnp.float32)
    r = jnp.reshape(flat, (fb, k, tw))
    gf = r[:, :, 0:128]
    gx = r[:, :, 128:131]
    ap = ap_ref[...]
    a = ap[:, None, 0:128]
    p = ap[:, None, 128:256]
    xq = xyzT_ref[0][:, None, 0:3]
    diff = xq - gx
    d = jnp.sqrt(jnp.sum(diff * diff, axis=2, keepdims=True))
    f = jnp.maximum(a + gf, 0.0)
    g = jnp.maximum(p + jnp.reshape(q, (fb, k, 128))
                    + w0_ref[...][None] * d, 0.0)
    out_ref[0] = jnp.max(f * g, axis=1)


def _final(g, ap, xyzT, w0, wgs):
    b, n, _ = xyzT.shape
    out_c = w0.shape[1]
    fb = 256
    nb = n // fb
    body = functools.partial(_final_body, fb=fb, k=KNN, tw=TW)
    return pl.pallas_call(
        body,
        grid=(b, nb),
        in_specs=[
            pl.BlockSpec((fb * KNN, TW), lambda i, j: (i * nb + j, 0)),
            pl.BlockSpec((fb, 2 * out_c), lambda i, j: (i * nb + j, 0)),
            pl.BlockSpec((1, fb, 4), lambda i, j: (i, j, 0)),
            pl.BlockSpec((1, out_c), lambda i, j: (0, 0)),
            pl.BlockSpec((4, out_c), lambda i, j: (0, 0)),
        ],
        out_specs=pl.BlockSpec((1, fb, out_c), lambda i, j: (i, j, 0)),
        out_shape=jax.ShapeDtypeStruct((b, n, out_c), jnp.float32),
        compiler_params=pltpu.CompilerParams(
            dimension_semantics=("parallel", "parallel")),
    )(g, ap, xyzT, w0, wgs)


def kernel(xyz, xyz_s, fea, fea_s, Wf, bf, Wg, bg):
    b, c, n = fea.shape
    out_c = Wf.shape[0]
    pad = jnp.zeros((b, n, 1), jnp.float32)
    xyzT = jnp.concatenate([jnp.swapaxes(xyz, 1, 2), pad], axis=2)
    xyz_sT = jnp.concatenate([jnp.swapaxes(xyz_s, 1, 2), pad], axis=2)
    wf1 = Wf[:, :c].T
    wf2 = Wf[:, c:].T
    wpad = jnp.zeros((1, out_c), jnp.float32)
    wge = jnp.concatenate([(Wg[:, 1:4] + Wg[:, 7:10]).T, wpad], axis=0)
    wgs = jnp.concatenate([(Wg[:, 4:7] - Wg[:, 7:10]).T, wpad], axis=0)
    w0 = Wg[:, 0:1].T
    bf2 = bf.reshape(1, out_c)
    bg2 = bg.reshape(1, out_c)

    table, ap = _prep(fea, fea_s, xyzT, xyz_sT, wf1, wf2, wge, bf2, bg2)
    idx = _knn(xyzT, xyz_s)
    gathered = _gather_rows(table, idx.reshape(-1))
    out = _final(gathered, ap, xyzT, w0, wgs)
    return jnp.swapaxes(out, 1, 2)


# speedup vs baseline: 1.6932x; 1.1024x over previous
import functools

import jax
import jax.numpy as jnp
from jax import lax
from jax.experimental import pallas as pl
from jax.experimental.pallas import tpu as pltpu
from jax.experimental.pallas import tpu_sc as plsc

KNN = 16
TW = 256
NEG_INF = float("-inf")

SC_NC = 2
SC_NS = 16
SC_NW = SC_NC * SC_NS


def _prep_body(fea_ref, fea_s_ref, xyzT_ref, xyz_sT_ref, wf1_ref, wf2_ref,
               wge_ref, bf_ref, bg_ref, t_ref, ap_ref):
    fea = fea_ref[0]
    fea_s = fea_s_ref[0]
    xq = xyzT_ref[0]
    xs = xyz_sT_ref[0]
    dn = (((0,), (0,)), ((), ()))
    a = lax.dot_general(fea, wf1_ref[...], dn,
                        preferred_element_type=jnp.float32)
    ap_ref[:, 0:128] = a + bf_ref[...]
    p = jnp.dot(xq, wge_ref[...], preferred_element_type=jnp.float32)
    ap_ref[:, 128:256] = p + bg_ref[...]
    bm = lax.dot_general(fea_s, wf2_ref[...], dn,
                         preferred_element_type=jnp.float32)
    t_ref[:, 0:128] = bm
    t_ref[:, 128:132] = xs
    t_ref[:, 132:256] = jnp.zeros_like(t_ref[:, 132:256])


def _prep(fea, fea_s, xyzT, xyz_sT, wf1, wf2, wge, bf2, bg2):
    b, c, n = fea.shape
    pb = 512
    nb = n // pb
    grid = (b, nb)
    out_c = wf1.shape[1]
    return pl.pallas_call(
        _prep_body,
        grid=grid,
        in_specs=[
            pl.BlockSpec((1, c, pb), lambda i, j: (i, 0, j)),
            pl.BlockSpec((1, c, pb), lambda i, j: (i, 0, j)),
            pl.BlockSpec((1, pb, 4), lambda i, j: (i, j, 0)),
            pl.BlockSpec((1, pb, 4), lambda i, j: (i, j, 0)),
            pl.BlockSpec((c, out_c), lambda i, j: (0, 0)),
            pl.BlockSpec((c, out_c), lambda i, j: (0, 0)),
            pl.BlockSpec((4, out_c), lambda i, j: (0, 0)),
            pl.BlockSpec((1, out_c), lambda i, j: (0, 0)),
            pl.BlockSpec((1, out_c), lambda i, j: (0, 0)),
        ],
        out_specs=[
            pl.BlockSpec((pb, TW), lambda i, j: (i * nb + j, 0)),
            pl.BlockSpec((pb, 2 * out_c), lambda i, j: (i * nb + j, 0)),
        ],
        out_shape=[
            jax.ShapeDtypeStruct((b * n, TW), jnp.float32),
            jax.ShapeDtypeStruct((b * n, 2 * out_c), jnp.float32),
        ],
        compiler_params=pltpu.CompilerParams(
            dimension_semantics=("parallel", "parallel")),
    )(fea, fea_s, xyzT, xyz_sT, wf1, wf2, wge, bf2, bg2)


def _ce(va, ia, vb, ib):
    c = vb > va
    return (jnp.maximum(va, vb), jnp.where(c, ib, ia),
            jnp.minimum(va, vb), jnp.where(c, ia, ib))


def _sort16_desc(v, ix):
    k = 2
    while k <= 16:
        j = k // 2
        while j >= 1:
            for i in range(16):
                l = i ^ j
                if l > i:
                    if (i & k) == 0:
                        v[i], ix[i], v[l], ix[l] = _ce(v[i], ix[i],
                                                       v[l], ix[l])
                    else:
                        v[l], ix[l], v[i], ix[i] = _ce(v[l], ix[l],
                                                       v[i], ix[i])
            j //= 2
        k *= 2
    return v, ix


def _merge16_desc(av, ai, bv, bi):
    mv, mi = [], []
    for i in range(16):
        c = bv[15 - i] > av[i]
        mv.append(jnp.maximum(av[i], bv[15 - i]))
        mi.append(jnp.where(c, bi[15 - i], ai[i]))
    for j in (8, 4, 2, 1):
        for i in range(16):
            l = i ^ j
            if l > i:
                mv[i], mi[i], mv[l], mi[l] = _ce(mv[i], mi[i], mv[l], mi[l])
    return mv, mi


def _knn_body(xyzT_ref, xyz_s_ref, idx_ref, *, n, qb, k):
    bi = pl.program_id(0)
    xq = xyzT_ref[0]
    xs = xyz_s_ref[0]
    e = jax.lax.dot_general(
        xq[:, 0:3].astype(jnp.bfloat16), xs.astype(jnp.bfloat16),
        (((1,), (0,)), ((), ())), preferred_element_type=jnp.float32)
    inner = -2.0 * e
    a2 = (xq[:, 0:1] * xq[:, 0:1] + xq[:, 1:2] * xq[:, 1:2]) \
        + xq[:, 2:3] * xq[:, 2:3]
    b2 = (xs[0:1, :] * xs[0:1, :] + xs[1:2, :] * xs[1:2, :]) \
        + xs[2:3, :] * xs[2:3, :]
    dis = (-a2 - inner) - b2

    nw = n // 128
    lane = lax.broadcasted_iota(jnp.int32, (qb, 128), 1)
    ninf = jnp.full((qb, 128), NEG_INF, jnp.float32)
    groups = []
    for g0 in range(0, nw, 16):
        gv, gi = [], []
        for i in range(16):
            if g0 + i < nw:
                gv.append(dis[:, (g0 + i) * 128:(g0 + i + 1) * 128])
                gi.append(lane + (g0 + i) * 128)
            else:
                gv.append(ninf)
                gi.append(lane)
        groups.append(_sort16_desc(gv, gi))
    while len(groups) > 1:
        nxt = []
        for a in range(0, len(groups), 2):
            nxt.append(_merge16_desc(groups[a][0], groups[a][1],
                                     groups[a + 1][0], groups[a + 1][1]))
        groups = nxt
    wv, wi = groups[0]

    cols = []
    for _ in range(k):
        m = jnp.max(wv[0], axis=1, keepdims=True)
        lsel = jnp.min(jnp.where(wv[0] == m, lane, jnp.int32(128)),
                       axis=1, keepdims=True)
        hit = lane == lsel
        cols.append(jnp.sum(jnp.where(hit, wi[0], 0), axis=1, keepdims=True))
        for t in range(15):
            wv[t] = jnp.where(hit, wv[t + 1], wv[t])
            wi[t] = jnp.where(hit, wi[t + 1], wi[t])
        wv[15] = jnp.where(hit, NEG_INF, wv[15])
    idx_ref[0] = jnp.concatenate(cols, axis=1) + bi * n


def _knn(xyzT, xyz_s):
    b, n, _ = xyzT.shape
    qb = 128
    grid = (b, n // qb)
    body = functools.partial(_knn_body, n=n, qb=qb, k=KNN)
    return pl.pallas_call(
        body,
        grid=grid,
        in_specs=[
            pl.BlockSpec((1, qb, 4), lambda i, j: (i, j, 0)),
            pl.BlockSpec((1, 3, n), lambda i, j: (i, 0, 0)),
        ],
        out_specs=pl.BlockSpec((1, qb, KNN), lambda i, j: (i, j, 0)),
        out_shape=jax.ShapeDtypeStruct((b, n, KNN), jnp.int32),
        compiler_params=pltpu.CompilerParams(
            dimension_semantics=("parallel", "parallel")),
    )(xyzT, xyz_s)


def _gather_rows(table, idx):
    m = idx.shape[0]
    tw = table.shape[1]
    per_w = m // SC_NW
    ch = 128
    n_ch = per_w // ch
    mesh = plsc.VectorSubcoreMesh(core_axis_name="c", subcore_axis_name="s")

    @functools.partial(
        pl.kernel,
        out_type=jax.ShapeDtypeStruct((m, tw), jnp.float32),
        mesh=mesh,
        scratch_types=[
            pltpu.VMEM((ch,), jnp.int32),
            pltpu.VMEM((ch, tw), jnp.float32),
            pltpu.SemaphoreType.DMA,
        ],
    )
    def gather_k(t_hbm, idx_hbm, out_hbm, idx_v, rows_v, sem):
        wid = lax.axis_index("s") * SC_NC + lax.axis_index("c")
        base = wid * per_w

        def body(i, carry):
            off = base + i * ch
            pltpu.sync_copy(idx_hbm.at[pl.ds(off, ch)], idx_v)
            pltpu.async_copy(t_hbm.at[idx_v], rows_v, sem).wait()
            pltpu.sync_copy(rows_v, out_hbm.at[pl.ds(off, ch)])
            return carry

        lax.fori_loop(0, n_ch, body, 0)

    return gather_k(table, idx)


def _final_body(g_ref, ap_ref, xyzT_ref, w0_ref, wgs_ref, out_ref,
                *, fb, k, tw):
    flat = g_ref[...]
    q = jnp.dot(flat[:, 128:132], wgs_ref[...],
                preferred_element_type=jnp.float32)
    r = jnp.reshape(flat, (fb, k, tw))
    gf = r[:, :, 0:128]
    gx = r[:, :, 128:131]
    ap = ap_ref[...]
    a = ap[:, None, 0:128]
    p = ap[:, None, 128:256]
    xq = xyzT_ref[0][:, None, 0:3]
    diff = xq - gx
    d = jnp.sqrt(jnp.sum(diff * diff, axis=2, keepdims=True))
    f = jnp.maximum(a + gf, 0.0)
    g = jnp.maximum(p + jnp.reshape(q, (fb, k, 128))
                    + w0_ref[...][None] * d, 0.0)
    out_ref[0] = jnp.max(f * g, axis=1)


def _final(g, ap, xyzT, w0, wgs):
    b, n, _ = xyzT.shape
    out_c = w0.shape[1]
    fb = 256
    nb = n // fb
    body = functools.partial(_final_body, fb=fb, k=KNN, tw=TW)
    return pl.pallas_call(
        body,
        grid=(b, nb),
        in_specs=[
            pl.BlockSpec((fb * KNN, TW), lambda i, j: (i * nb + j, 0)),
            pl.BlockSpec((fb, 2 * out_c), lambda i, j: (i * nb + j, 0)),
            pl.BlockSpec((1, fb, 4), lambda i, j: (i, j, 0)),
            pl.BlockSpec((1, out_c), lambda i, j: (0, 0)),
            pl.BlockSpec((4, out_c), lambda i, j: (0, 0)),
        ],
        out_specs=pl.BlockSpec((1, fb, out_c), lambda i, j: (i, j, 0)),
        out_shape=jax.ShapeDtypeStruct((b, n, out_c), jnp.float32),
        compiler_params=pltpu.CompilerParams(
            dimension_semantics=("parallel", "parallel")),
    )(g, ap, xyzT, w0, wgs)


def kernel(xyz, xyz_s, fea, fea_s, Wf, bf, Wg, bg):
    b, c, n = fea.shape
    out_c = Wf.shape[0]
    pad = jnp.zeros((b, n, 1), jnp.float32)
    xyzT = jnp.concatenate([jnp.swapaxes(xyz, 1, 2), pad], axis=2)
    xyz_sT = jnp.concatenate([jnp.swapaxes(xyz_s, 1, 2), pad], axis=2)
    wf1 = Wf[:, :c].T
    wf2 = Wf[:, c:].T
    wpad = jnp.zeros((1, out_c), jnp.float32)
    wge = jnp.concatenate([(Wg[:, 1:4] + Wg[:, 7:10]).T, wpad], axis=0)
    wgs = jnp.concatenate([(Wg[:, 4:7] - Wg[:, 7:10]).T, wpad], axis=0)
    w0 = Wg[:, 0:1].T
    bf2 = bf.reshape(1, out_c)
    bg2 = bg.reshape(1, out_c)

    table, ap = _prep(fea, fea_s, xyzT, xyz_sT, wf1, wf2, wge, bf2, bg2)
    idx = _knn(xyzT, xyz_s)
    gathered = _gather_rows(table, idx.reshape(-1))
    out = _final(gathered, ap, xyzT, w0, wgs)
    return jnp.swapaxes(out, 1, 2)


# speedup vs baseline: 1.7622x; 1.0408x over previous
import functools

import jax
import jax.numpy as jnp
from jax import lax
from jax.experimental import pallas as pl
from jax.experimental.pallas import tpu as pltpu
from jax.experimental.pallas import tpu_sc as plsc

KNN = 16
TW = 256
NEG_INF = float("-inf")

SC_NC = 2
SC_NS = 16
SC_NW = SC_NC * SC_NS


def _prep_body(fea_ref, fea_s_ref, xyzT_ref, xyz_sT_ref, wf1_ref, wf2_ref,
               wge_ref, bf_ref, bg_ref, t_ref, ap_ref):
    fea = fea_ref[0]
    fea_s = fea_s_ref[0]
    xq = xyzT_ref[0]
    xs = xyz_sT_ref[0]
    dn = (((0,), (0,)), ((), ()))
    a = lax.dot_general(fea, wf1_ref[...], dn,
                        preferred_element_type=jnp.float32)
    ap_ref[:, 0:128] = a + bf_ref[...]
    p = jnp.dot(xq, wge_ref[...], preferred_element_type=jnp.float32)
    ap_ref[:, 128:256] = p + bg_ref[...]
    bm = lax.dot_general(fea_s, wf2_ref[...], dn,
                         preferred_element_type=jnp.float32)
    t_ref[:, 0:128] = bm
    t_ref[:, 128:132] = xs
    t_ref[:, 132:256] = jnp.zeros_like(t_ref[:, 132:256])


def _prep(fea, fea_s, xyzT, xyz_sT, wf1, wf2, wge, bf2, bg2):
    b, c, n = fea.shape
    pb = 512
    nb = n // pb
    grid = (b, nb)
    out_c = wf1.shape[1]
    return pl.pallas_call(
        _prep_body,
        grid=grid,
        in_specs=[
            pl.BlockSpec((1, c, pb), lambda i, j: (i, 0, j)),
            pl.BlockSpec((1, c, pb), lambda i, j: (i, 0, j)),
            pl.BlockSpec((1, pb, 4), lambda i, j: (i, j, 0)),
            pl.BlockSpec((1, pb, 4), lambda i, j: (i, j, 0)),
            pl.BlockSpec((c, out_c), lambda i, j: (0, 0)),
            pl.BlockSpec((c, out_c), lambda i, j: (0, 0)),
            pl.BlockSpec((4, out_c), lambda i, j: (0, 0)),
            pl.BlockSpec((1, out_c), lambda i, j: (0, 0)),
            pl.BlockSpec((1, out_c), lambda i, j: (0, 0)),
        ],
        out_specs=[
            pl.BlockSpec((pb, TW), lambda i, j: (i * nb + j, 0)),
            pl.BlockSpec((pb, 2 * out_c), lambda i, j: (i * nb + j, 0)),
        ],
        out_shape=[
            jax.ShapeDtypeStruct((b * n, TW), jnp.float32),
            jax.ShapeDtypeStruct((b * n, 2 * out_c), jnp.float32),
        ],
        compiler_params=pltpu.CompilerParams(
            dimension_semantics=("parallel", "parallel")),
    )(fea, fea_s, xyzT, xyz_sT, wf1, wf2, wge, bf2, bg2)


def _ce(va, ia, vb, ib):
    c = vb > va
    return (jnp.maximum(va, vb), jnp.where(c, ib, ia),
            jnp.minimum(va, vb), jnp.where(c, ia, ib))


def _sort16_desc(v, ix):
    k = 2
    while k <= 16:
        j = k // 2
        while j >= 1:
            for i in range(16):
                l = i ^ j
                if l > i:
                    if (i & k) == 0:
                        v[i], ix[i], v[l], ix[l] = _ce(v[i], ix[i],
                                                       v[l], ix[l])
                    else:
                        v[l], ix[l], v[i], ix[i] = _ce(v[l], ix[l],
                                                       v[i], ix[i])
            j //= 2
        k *= 2
    return v, ix


def _merge16_desc(av, ai, bv, bi):
    mv, mi = [], []
    for i in range(16):
        c = bv[15 - i] > av[i]
        mv.append(jnp.maximum(av[i], bv[15 - i]))
        mi.append(jnp.where(c, bi[15 - i], ai[i]))
    for j in (8, 4, 2, 1):
        for i in range(16):
            l = i ^ j
            if l > i:
                mv[i], mi[i], mv[l], mi[l] = _ce(mv[i], mi[i], mv[l], mi[l])
    return mv, mi


def _knn_body(xyzT_ref, xyz_s_ref, idx_ref, *, n, qb, k):
    bi = pl.program_id(0)
    xq = xyzT_ref[0]
    xs = xyz_s_ref[0]
    e = jax.lax.dot_general(
        xq[:, 0:3].astype(jnp.bfloat16), xs.astype(jnp.bfloat16),
        (((1,), (0,)), ((), ())), preferred_element_type=jnp.float32)
    inner = -2.0 * e
    a2 = (xq[:, 0:1] * xq[:, 0:1] + xq[:, 1:2] * xq[:, 1:2]) \
        + xq[:, 2:3] * xq[:, 2:3]
    b2 = (xs[0:1, :] * xs[0:1, :] + xs[1:2, :] * xs[1:2, :]) \
        + xs[2:3, :] * xs[2:3, :]
    dis = (-a2 - inner) - b2

    nw = n // 128
    lane = lax.broadcasted_iota(jnp.int32, (qb, 128), 1)
    ninf = jnp.full((qb, 128), NEG_INF, jnp.float32)
    groups = []
    for g0 in range(0, nw, 16):
        gv, gi = [], []
        for i in range(16):
            if g0 + i < nw:
                gv.append(dis[:, (g0 + i) * 128:(g0 + i + 1) * 128])
                gi.append(lane + (g0 + i) * 128)
            else:
                gv.append(ninf)
                gi.append(lane)
        groups.append(_sort16_desc(gv, gi))
    while len(groups) > 1:
        nxt = []
        for a in range(0, len(groups), 2):
            nxt.append(_merge16_desc(groups[a][0], groups[a][1],
                                     groups[a + 1][0], groups[a + 1][1]))
        groups = nxt
    wv, wi = groups[0]

    cols = []
    for _ in range(k):
        m = jnp.max(wv[0], axis=1, keepdims=True)
        lsel = jnp.min(jnp.where(wv[0] == m, lane, jnp.int32(128)),
                       axis=1, keepdims=True)
        hit = lane == lsel
        cols.append(jnp.sum(jnp.where(hit, wi[0], 0), axis=1, keepdims=True))
        for t in range(15):
            wv[t] = jnp.where(hit, wv[t + 1], wv[t])
            wi[t] = jnp.where(hit, wi[t + 1], wi[t])
        wv[15] = jnp.where(hit, NEG_INF, wv[15])
    idx_ref[0] = jnp.concatenate(cols, axis=1) + bi * n


def _knn(xyzT, xyz_s):
    b, n, _ = xyzT.shape
    qb = 128
    grid = (b, n // qb)
    body = functools.partial(_knn_body, n=n, qb=qb, k=KNN)
    return pl.pallas_call(
        body,
        grid=grid,
        in_specs=[
            pl.BlockSpec((1, qb, 4), lambda i, j: (i, j, 0)),
            pl.BlockSpec((1, 3, n), lambda i, j: (i, 0, 0)),
        ],
        out_specs=pl.BlockSpec((1, qb, KNN), lambda i, j: (i, j, 0)),
        out_shape=jax.ShapeDtypeStruct((b, n, KNN), jnp.int32),
        compiler_params=pltpu.CompilerParams(
            dimension_semantics=("parallel", "parallel")),
    )(xyzT, xyz_s)


def _gather_rows(table, idx):
    m = idx.shape[0]
    tw = table.shape[1]
    per_w = m // SC_NW
    ch = 128
    n_ch = per_w // ch
    mesh = plsc.VectorSubcoreMesh(core_axis_name="c", subcore_axis_name="s")

    @functools.partial(
        pl.kernel,
        out_type=jax.ShapeDtypeStruct((m, tw), jnp.float32),
        mesh=mesh,
        scratch_types=[
            pltpu.VMEM((2, ch), jnp.int32),
            pltpu.VMEM((2, ch, tw), jnp.float32),
            pltpu.SemaphoreType.DMA,
            pltpu.SemaphoreType.DMA,
        ],
    )
    def gather_k(t_hbm, idx_hbm, out_hbm, idx_v, rows_v, sem0, sem1):
        wid = lax.axis_index("s") * SC_NC + lax.axis_index("c")
        base = wid * per_w
        sems = (sem0, sem1)

        def start(i, buf):
            pltpu.sync_copy(idx_hbm.at[pl.ds(base + i * ch, ch)],
                            idx_v.at[buf])
            pltpu.async_copy(t_hbm.at[idx_v.at[buf]], rows_v.at[buf],
                             sems[buf])

        def drain_store(i, buf):
            pltpu.make_async_copy(t_hbm.at[idx_v.at[buf]], rows_v.at[buf],
                                  sems[buf]).wait()
            pltpu.sync_copy(rows_v.at[buf], out_hbm.at[pl.ds(base + i * ch,
                                                             ch)])

        start(0, 0)

        def body(ii, carry):
            i0 = ii * 2
            for b2 in range(2):
                i = i0 + b2
                nxt = i + 1

                @pl.when(nxt < n_ch)
                def _():
                    start(nxt, (b2 + 1) % 2)

                drain_store(i, b2)
            return carry

        lax.fori_loop(0, n_ch // 2, body, 0)

    return gather_k(table, idx)


def _final_body(g_ref, ap_ref, xyzT_ref, w0_ref, wgs_ref, out_ref,
                *, fb, k, tw):
    flat = g_ref[...]
    q = jnp.dot(flat[:, 128:132], wgs_ref[...],
                preferred_element_type=jnp.float32)
    r = jnp.reshape(flat, (fb, k, tw))
    gf = r[:, :, 0:128]
    gx = r[:, :, 128:131]
    ap = ap_ref[...]
    a = ap[:, None, 0:128]
    p = ap[:, None, 128:256]
    xq = xyzT_ref[0][:, None, 0:3]
    diff = xq - gx
    d = jnp.sqrt(jnp.sum(diff * diff, axis=2, keepdims=True))
    f = jnp.maximum(a + gf, 0.0)
    g = jnp.maximum(p + jnp.reshape(q, (fb, k, 128))
                    + w0_ref[...][None] * d, 0.0)
    out_ref[0] = jnp.max(f * g, axis=1)


def _final(g, ap, xyzT, w0, wgs):
    b, n, _ = xyzT.shape
    out_c = w0.shape[1]
    fb = 256
    nb = n // fb
    body = functools.partial(_final_body, fb=fb, k=KNN, tw=TW)
    return pl.pallas_call(
        body,
        grid=(b, nb),
        in_specs=[
            pl.BlockSpec((fb * KNN, TW), lambda i, j: (i * nb + j, 0)),
            pl.BlockSpec((fb, 2 * out_c), lambda i, j: (i * nb + j, 0)),
            pl.BlockSpec((1, fb, 4), lambda i, j: (i, j, 0)),
            pl.BlockSpec((1, out_c), lambda i, j: (0, 0)),
            pl.BlockSpec((4, out_c), lambda i, j: (0, 0)),
        ],
        out_specs=pl.BlockSpec((1, fb, out_c), lambda i, j: (i, j, 0)),
        out_shape=jax.ShapeDtypeStruct((b, n, out_c), jnp.float32),
        compiler_params=pltpu.CompilerParams(
            dimension_semantics=("parallel", "parallel")),
    )(g, ap, xyzT, w0, wgs)


def kernel(xyz, xyz_s, fea, fea_s, Wf, bf, Wg, bg):
    b, c, n = fea.shape
    out_c = Wf.shape[0]
    pad = jnp.zeros((b, n, 1), jnp.float32)
    xyzT = jnp.concatenate([jnp.swapaxes(xyz, 1, 2), pad], axis=2)
    xyz_sT = jnp.concatenate([jnp.swapaxes(xyz_s, 1, 2), pad], axis=2)
    wf1 = Wf[:, :c].T
    wf2 = Wf[:, c:].T
    wpad = jnp.zeros((1, out_c), jnp.float32)
    wge = jnp.concatenate([(Wg[:, 1:4] + Wg[:, 7:10]).T, wpad], axis=0)
    wgs = jnp.concatenate([(Wg[:, 4:7] - Wg[:, 7:10]).T, wpad], axis=0)
    w0 = Wg[:, 0:1].T
    bf2 = bf.reshape(1, out_c)
    bg2 = bg.reshape(1, out_c)

    table, ap = _prep(fea, fea_s, xyzT, xyz_sT, wf1, wf2, wge, bf2, bg2)
    idx = _knn(xyzT, xyz_s)
    gathered = _gather_rows(table, idx.reshape(-1))
    out = _final(gathered, ap, xyzT, w0, wgs)
    return jnp.swapaxes(out, 1, 2)


# speedup vs baseline: 1.7657x; 1.0020x over previous
import functools

import jax
import jax.numpy as jnp
from jax import lax
from jax.experimental import pallas as pl
from jax.experimental.pallas import tpu as pltpu
from jax.experimental.pallas import tpu_sc as plsc

KNN = 16
TW = 256
NEG_INF = float("-inf")

SC_NC = 2
SC_NS = 16
SC_NW = SC_NC * SC_NS


def _prep_body(fea_ref, fea_s_ref, xyzT_ref, xyz_sT_ref, wf1_ref, wf2_ref,
               wge_ref, bf_ref, bg_ref, t_ref, ap_ref):
    fea = fea_ref[0]
    fea_s = fea_s_ref[0]
    xq = xyzT_ref[0]
    xs = xyz_sT_ref[0]
    dn = (((0,), (0,)), ((), ()))
    a = lax.dot_general(fea, wf1_ref[...], dn,
                        preferred_element_type=jnp.float32)
    ap_ref[:, 0:128] = a + bf_ref[...]
    p = jnp.dot(xq, wge_ref[...], preferred_element_type=jnp.float32)
    ap_ref[:, 128:256] = p + bg_ref[...]
    bm = lax.dot_general(fea_s, wf2_ref[...], dn,
                         preferred_element_type=jnp.float32)
    t_ref[:, 0:128] = bm
    t_ref[:, 128:132] = xs
    t_ref[:, 132:256] = jnp.zeros_like(t_ref[:, 132:256])


def _prep(fea, fea_s, xyzT, xyz_sT, wf1, wf2, wge, bf2, bg2):
    b, c, n = fea.shape
    pb = 512
    nb = n // pb
    grid = (b, nb)
    out_c = wf1.shape[1]
    return pl.pallas_call(
        _prep_body,
        grid=grid,
        in_specs=[
            pl.BlockSpec((1, c, pb), lambda i, j: (i, 0, j)),
            pl.BlockSpec((1, c, pb), lambda i, j: (i, 0, j)),
            pl.BlockSpec((1, pb, 4), lambda i, j: (i, j, 0)),
            pl.BlockSpec((1, pb, 4), lambda i, j: (i, j, 0)),
            pl.BlockSpec((c, out_c), lambda i, j: (0, 0)),
            pl.BlockSpec((c, out_c), lambda i, j: (0, 0)),
            pl.BlockSpec((4, out_c), lambda i, j: (0, 0)),
            pl.BlockSpec((1, out_c), lambda i, j: (0, 0)),
            pl.BlockSpec((1, out_c), lambda i, j: (0, 0)),
        ],
        out_specs=[
            pl.BlockSpec((pb, TW), lambda i, j: (i * nb + j, 0)),
            pl.BlockSpec((pb, 2 * out_c), lambda i, j: (i * nb + j, 0)),
        ],
        out_shape=[
            jax.ShapeDtypeStruct((b * n, TW), jnp.float32),
            jax.ShapeDtypeStruct((b * n, 2 * out_c), jnp.float32),
        ],
        compiler_params=pltpu.CompilerParams(
            dimension_semantics=("parallel", "parallel")),
    )(fea, fea_s, xyzT, xyz_sT, wf1, wf2, wge, bf2, bg2)


def _ce(va, ia, vb, ib):
    c = vb > va
    return (jnp.maximum(va, vb), jnp.where(c, ib, ia),
            jnp.minimum(va, vb), jnp.where(c, ia, ib))


def _sort16_desc(v, ix):
    k = 2
    while k <= 16:
        j = k // 2
        while j >= 1:
            for i in range(16):
                l = i ^ j
                if l > i:
                    if (i & k) == 0:
                        v[i], ix[i], v[l], ix[l] = _ce(v[i], ix[i],
                                                       v[l], ix[l])
                    else:
                        v[l], ix[l], v[i], ix[i] = _ce(v[l], ix[l],
                                                       v[i], ix[i])
            j //= 2
        k *= 2
    return v, ix


def _merge16_desc(av, ai, bv, bi):
    mv, mi = [], []
    for i in range(16):
        c = bv[15 - i] > av[i]
        mv.append(jnp.maximum(av[i], bv[15 - i]))
        mi.append(jnp.where(c, bi[15 - i], ai[i]))
    for j in (8, 4, 2, 1):
        for i in range(16):
            l = i ^ j
            if l > i:
                mv[i], mi[i], mv[l], mi[l] = _ce(mv[i], mi[i], mv[l], mi[l])
    return mv, mi


def _knn_body(xyzT_ref, xyz_s_ref, idx_ref, *, n, qb, k):
    bi = pl.program_id(0)
    xq = xyzT_ref[0]
    xs = xyz_s_ref[0]
    e = jax.lax.dot_general(
        xq[:, 0:3].astype(jnp.bfloat16), xs.astype(jnp.bfloat16),
        (((1,), (0,)), ((), ())), preferred_element_type=jnp.float32)
    inner = -2.0 * e
    a2 = (xq[:, 0:1] * xq[:, 0:1] + xq[:, 1:2] * xq[:, 1:2]) \
        + xq[:, 2:3] * xq[:, 2:3]
    b2 = (xs[0:1, :] * xs[0:1, :] + xs[1:2, :] * xs[1:2, :]) \
        + xs[2:3, :] * xs[2:3, :]
    dis = (-a2 - inner) - b2

    nw = n // 128
    lane = lax.broadcasted_iota(jnp.int32, (qb, 128), 1)
    ninf = jnp.full((qb, 128), NEG_INF, jnp.float32)
    groups = []
    for g0 in range(0, nw, 16):
        gv, gi = [], []
        for i in range(16):
            if g0 + i < nw:
                gv.append(dis[:, (g0 + i) * 128:(g0 + i + 1) * 128])
                gi.append(lane + (g0 + i) * 128)
            else:
                gv.append(ninf)
                gi.append(lane)
        groups.append(_sort16_desc(gv, gi))
    while len(groups) > 1:
        nxt = []
        for a in range(0, len(groups), 2):
            nxt.append(_merge16_desc(groups[a][0], groups[a][1],
                                     groups[a + 1][0], groups[a + 1][1]))
        groups = nxt
    wv, wi = groups[0]

    cols = []
    for t in range(k):
        m = jnp.max(wv[0], axis=1, keepdims=True)
        lsel = jnp.min(jnp.where(wv[0] == m, lane, jnp.int32(128)),
                       axis=1, keepdims=True)
        hit = lane == lsel
        cols.append(jnp.sum(jnp.where(hit, wi[0], 0), axis=1, keepdims=True))
        for j in range(15 - t):
            wv[j] = jnp.where(hit, wv[j + 1], wv[j])
            wi[j] = jnp.where(hit, wi[j + 1], wi[j])
        if t < 15:
            wv[15 - t] = jnp.where(hit, NEG_INF, wv[15 - t])
    idx_ref[0] = jnp.concatenate(cols, axis=1) + bi * n


def _knn(xyzT, xyz_s):
    b, n, _ = xyzT.shape
    qb = 128
    grid = (b, n // qb)
    body = functools.partial(_knn_body, n=n, qb=qb, k=KNN)
    return pl.pallas_call(
        body,
        grid=grid,
        in_specs=[
            pl.BlockSpec((1, qb, 4), lambda i, j: (i, j, 0)),
            pl.BlockSpec((1, 3, n), lambda i, j: (i, 0, 0)),
        ],
        out_specs=pl.BlockSpec((1, qb, KNN), lambda i, j: (i, j, 0)),
        out_shape=jax.ShapeDtypeStruct((b, n, KNN), jnp.int32),
        compiler_params=pltpu.CompilerParams(
            dimension_semantics=("parallel", "parallel")),
    )(xyzT, xyz_s)


def _gather_rows(table, idx):
    m = idx.shape[0]
    tw = table.shape[1]
    per_w = m // SC_NW
    ch = 128
    n_ch = per_w // ch
    mesh = plsc.VectorSubcoreMesh(core_axis_name="c", subcore_axis_name="s")

    @functools.partial(
        pl.kernel,
        out_type=jax.ShapeDtypeStruct((m, tw), jnp.float32),
        mesh=mesh,
        scratch_types=[
            pltpu.VMEM((2, ch), jnp.int32),
            pltpu.VMEM((2, ch, tw), jnp.float32),
            pltpu.SemaphoreType.DMA,
            pltpu.SemaphoreType.DMA,
        ],
    )
    def gather_k(t_hbm, idx_hbm, out_hbm, idx_v, rows_v, sem0, sem1):
        wid = lax.axis_index("s") * SC_NC + lax.axis_index("c")
        base = wid * per_w
        sems = (sem0, sem1)

        def start(i, buf):
            pltpu.sync_copy(idx_hbm.at[pl.ds(base + i * ch, ch)],
                            idx_v.at[buf])
            pltpu.async_copy(t_hbm.at[idx_v.at[buf]], rows_v.at[buf],
                             sems[buf])

        def drain_store(i, buf):
            pltpu.make_async_copy(t_hbm.at[idx_v.at[buf]], rows_v.at[buf],
                                  sems[buf]).wait()
            pltpu.sync_copy(rows_v.at[buf], out_hbm.at[pl.ds(base + i * ch,
                                                             ch)])

        start(0, 0)

        def body(ii, carry):
            i0 = ii * 2
            for b2 in range(2):
                i = i0 + b2
                nxt = i + 1

                @pl.when(nxt < n_ch)
                def _():
                    start(nxt, (b2 + 1) % 2)

                drain_store(i, b2)
            return carry

        lax.fori_loop(0, n_ch // 2, body, 0)

    return gather_k(table, idx)


def _final_body(g_ref, ap_ref, xyzT_ref, w0_ref, wgs_ref, out_ref,
                *, fb, k, tw):
    flat = g_ref[...]
    q = jnp.dot(flat[:, 128:132], wgs_ref[...],
                preferred_element_type=jnp.float32)
    r = jnp.reshape(flat, (fb, k, tw))
    gf = r[:, :, 0:128]
    gx = r[:, :, 128:131]
    ap = ap_ref[...]
    a = ap[:, None, 0:128]
    p = ap[:, None, 128:256]
    xq = xyzT_ref[0][:, None, 0:3]
    diff = xq - gx
    d = jnp.sqrt(jnp.sum(diff * diff, axis=2, keepdims=True))
    f = jnp.maximum(a + gf, 0.0)
    g = jnp.maximum(p + jnp.reshape(q, (fb, k, 128))
                    + w0_ref[...][None] * d, 0.0)
    out_ref[0] = jnp.max(f * g, axis=1)


def _final(g, ap, xyzT, w0, wgs):
    b, n, _ = xyzT.shape
    out_c = w0.shape[1]
    fb = 256
    nb = n // fb
    body = functools.partial(_final_body, fb=fb, k=KNN, tw=TW)
    return pl.pallas_call(
        body,
        grid=(b, nb),
        in_specs=[
            pl.BlockSpec((fb * KNN, TW), lambda i, j: (i * nb + j, 0)),
            pl.BlockSpec((fb, 2 * out_c), lambda i, j: (i * nb + j, 0)),
            pl.BlockSpec((1, fb, 4), lambda i, j: (i, j, 0)),
            pl.BlockSpec((1, out_c), lambda i, j: (0, 0)),
            pl.BlockSpec((4, out_c), lambda i, j: (0, 0)),
        ],
        out_specs=pl.BlockSpec((1, fb, out_c), lambda i, j: (i, j, 0)),
        out_shape=jax.ShapeDtypeStruct((b, n, out_c), jnp.float32),
        compiler_params=pltpu.CompilerParams(
            dimension_semantics=("parallel", "parallel")),
    )(g, ap, xyzT, w0, wgs)


def kernel(xyz, xyz_s, fea, fea_s, Wf, bf, Wg, bg):
    b, c, n = fea.shape
    out_c = Wf.shape[0]
    pad = jnp.zeros((b, n, 1), jnp.float32)
    xyzT = jnp.concatenate([jnp.swapaxes(xyz, 1, 2), pad], axis=2)
    xyz_sT = jnp.concatenate([jnp.swapaxes(xyz_s, 1, 2), pad], axis=2)
    wf1 = Wf[:, :c].T
    wf2 = Wf[:, c:].T
    wpad = jnp.zeros((1, out_c), jnp.float32)
    wge = jnp.concatenate([(Wg[:, 1:4] + Wg[:, 7:10]).T, wpad], axis=0)
    wgs = jnp.concatenate([(Wg[:, 4:7] - Wg[:, 7:10]).T, wpad], axis=0)
    w0 = Wg[:, 0:1].T
    bf2 = bf.reshape(1, out_c)
    bg2 = bg.reshape(1, out_c)

    table, ap = _prep(fea, fea_s, xyzT, xyz_sT, wf1, wf2, wge, bf2, bg2)
    idx = _knn(xyzT, xyz_s)
    gathered = _gather_rows(table, idx.reshape(-1))
    out = _final(gathered, ap, xyzT, w0, wgs)
    return jnp.swapaxes(out, 1, 2)


# speedup vs baseline: 1.8601x; 1.0535x over previous
import functools

import jax
import jax.numpy as jnp
from jax import lax
from jax.experimental import pallas as pl
from jax.experimental.pallas import tpu as pltpu
from jax.experimental.pallas import tpu_sc as plsc

KNN = 16
TW = 256
NEG_INF = float("-inf")

SC_NC = 2
SC_NS = 16
SC_NW = SC_NC * SC_NS


def _prep_body(fea_ref, fea_s_ref, xyzT_ref, xyz_sT_ref, wf1_ref, wf2_ref,
               wge_ref, bf_ref, bg_ref, t_ref, ap_ref):
    fea = fea_ref[0]
    fea_s = fea_s_ref[0]
    xq = xyzT_ref[0]
    xs = xyz_sT_ref[0]
    dn = (((0,), (0,)), ((), ()))
    a = lax.dot_general(fea, wf1_ref[...], dn,
                        preferred_element_type=jnp.float32)
    ap_ref[:, 0:128] = a + bf_ref[...]
    p = jnp.dot(xq, wge_ref[...], preferred_element_type=jnp.float32)
    ap_ref[:, 128:256] = p + bg_ref[...]
    bm = lax.dot_general(fea_s, wf2_ref[...], dn,
                         preferred_element_type=jnp.float32)
    t_ref[:, 0:128] = bm
    t_ref[:, 128:132] = xs
    t_ref[:, 132:256] = jnp.zeros_like(t_ref[:, 132:256])


def _prep(fea, fea_s, xyzT, xyz_sT, wf1, wf2, wge, bf2, bg2):
    b, c, n = fea.shape
    pb = 512
    nb = n // pb
    grid = (b, nb)
    out_c = wf1.shape[1]
    return pl.pallas_call(
        _prep_body,
        grid=grid,
        in_specs=[
            pl.BlockSpec((1, c, pb), lambda i, j: (i, 0, j)),
            pl.BlockSpec((1, c, pb), lambda i, j: (i, 0, j)),
            pl.BlockSpec((1, pb, 4), lambda i, j: (i, j, 0)),
            pl.BlockSpec((1, pb, 4), lambda i, j: (i, j, 0)),
            pl.BlockSpec((c, out_c), lambda i, j: (0, 0)),
            pl.BlockSpec((c, out_c), lambda i, j: (0, 0)),
            pl.BlockSpec((4, out_c), lambda i, j: (0, 0)),
            pl.BlockSpec((1, out_c), lambda i, j: (0, 0)),
            pl.BlockSpec((1, out_c), lambda i, j: (0, 0)),
        ],
        out_specs=[
            pl.BlockSpec((pb, TW), lambda i, j: (i * nb + j, 0)),
            pl.BlockSpec((pb, 2 * out_c), lambda i, j: (i * nb + j, 0)),
        ],
        out_shape=[
            jax.ShapeDtypeStruct((b * n, TW), jnp.float32),
            jax.ShapeDtypeStruct((b * n, 2 * out_c), jnp.float32),
        ],
        compiler_params=pltpu.CompilerParams(
            dimension_semantics=("parallel", "parallel")),
    )(fea, fea_s, xyzT, xyz_sT, wf1, wf2, wge, bf2, bg2)


def _ce(va, ia, vb, ib):
    c = vb > va
    return (jnp.maximum(va, vb), jnp.where(c, ib, ia),
            jnp.minimum(va, vb), jnp.where(c, ia, ib))


def _sort16_desc(v, ix):
    k = 2
    while k <= 16:
        j = k // 2
        while j >= 1:
            for i in range(16):
                l = i ^ j
                if l > i:
                    if (i & k) == 0:
                        v[i], ix[i], v[l], ix[l] = _ce(v[i], ix[i],
                                                       v[l], ix[l])
                    else:
                        v[l], ix[l], v[i], ix[i] = _ce(v[l], ix[l],
                                                       v[i], ix[i])
            j //= 2
        k *= 2
    return v, ix


def _merge16_desc(av, ai, bv, bi):
    mv, mi = [], []
    for i in range(16):
        c = bv[15 - i] > av[i]
        mv.append(jnp.maximum(av[i], bv[15 - i]))
        mi.append(jnp.where(c, bi[15 - i], ai[i]))
    for j in (8, 4, 2, 1):
        for i in range(16):
            l = i ^ j
            if l > i:
                mv[i], mi[i], mv[l], mi[l] = _ce(mv[i], mi[i], mv[l], mi[l])
    return mv, mi


def _knn_body(xyzT_ref, xyz_s_ref, idx_ref, *, n, qb, k):
    bi = pl.program_id(0)
    xq = xyzT_ref[0]
    xs = xyz_s_ref[0]
    e = jax.lax.dot_general(
        xq[:, 0:3].astype(jnp.bfloat16), xs.astype(jnp.bfloat16),
        (((1,), (0,)), ((), ())), preferred_element_type=jnp.float32)
    inner = -2.0 * e
    a2 = (xq[:, 0:1] * xq[:, 0:1] + xq[:, 1:2] * xq[:, 1:2]) \
        + xq[:, 2:3] * xq[:, 2:3]
    b2 = (xs[0:1, :] * xs[0:1, :] + xs[1:2, :] * xs[1:2, :]) \
        + xs[2:3, :] * xs[2:3, :]
    dis = (-a2 - inner) - b2

    nw = n // 128
    lane = lax.broadcasted_iota(jnp.int32, (qb, 128), 1)
    ninf = jnp.full((qb, 128), NEG_INF, jnp.float32)
    groups = []
    for g0 in range(0, nw, 16):
        gv, gi = [], []
        for i in range(16):
            if g0 + i < nw:
                gv.append(dis[:, (g0 + i) * 128:(g0 + i + 1) * 128])
                gi.append(lane + (g0 + i) * 128)
            else:
                gv.append(ninf)
                gi.append(lane)
        groups.append(_sort16_desc(gv, gi))
    while len(groups) > 1:
        nxt = []
        for a in range(0, len(groups), 2):
            nxt.append(_merge16_desc(groups[a][0], groups[a][1],
                                     groups[a + 1][0], groups[a + 1][1]))
        groups = nxt
    wv, wi = groups[0]

    cols = []
    for t in range(k):
        m = jnp.max(wv[0], axis=1, keepdims=True)
        lsel = jnp.min(jnp.where(wv[0] == m, lane, jnp.int32(128)),
                       axis=1, keepdims=True)
        hit = lane == lsel
        cols.append(jnp.sum(jnp.where(hit, wi[0], 0), axis=1, keepdims=True))
        for j in range(15 - t):
            wv[j] = jnp.where(hit, wv[j + 1], wv[j])
            wi[j] = jnp.where(hit, wi[j + 1], wi[j])
        if t < 15:
            wv[15 - t] = jnp.where(hit, NEG_INF, wv[15 - t])
    idx_ref[0] = jnp.concatenate(cols, axis=1) + bi * n


def _knn(xyzT, xyz_s):
    b, n, _ = xyzT.shape
    qb = 128
    grid = (b, n // qb)
    body = functools.partial(_knn_body, n=n, qb=qb, k=KNN)
    return pl.pallas_call(
        body,
        grid=grid,
        in_specs=[
            pl.BlockSpec((1, qb, 4), lambda i, j: (i, j, 0)),
            pl.BlockSpec((1, 3, n), lambda i, j: (i, 0, 0)),
        ],
        out_specs=pl.BlockSpec((1, qb, KNN), lambda i, j: (i, j, 0)),
        out_shape=jax.ShapeDtypeStruct((b, n, KNN), jnp.int32),
        compiler_params=pltpu.CompilerParams(
            dimension_semantics=("parallel", "parallel")),
    )(xyzT, xyz_s)


def _gather_rows(table, idx):
    m = idx.shape[0]
    tw = table.shape[1]
    per_w = m // SC_NW
    ch = 128
    n_ch = per_w // ch
    mesh = plsc.VectorSubcoreMesh(core_axis_name="c", subcore_axis_name="s")

    @functools.partial(
        pl.kernel,
        out_type=jax.ShapeDtypeStruct((m, tw), jnp.float32),
        mesh=mesh,
        scratch_types=[
            pltpu.VMEM((2, ch), jnp.int32),
            pltpu.VMEM((2, ch, tw), jnp.float32),
            pltpu.SemaphoreType.DMA,
            pltpu.SemaphoreType.DMA,
        ],
    )
    def gather_k(t_hbm, idx_hbm, out_hbm, idx_v, rows_v, sem0, sem1):
        wid = lax.axis_index("s") * SC_NC + lax.axis_index("c")
        base = wid * per_w
        sems = (sem0, sem1)

        def start(i, buf):
            pltpu.sync_copy(idx_hbm.at[pl.ds(base + i * ch, ch)],
                            idx_v.at[buf])
            pltpu.async_copy(t_hbm.at[idx_v.at[buf]], rows_v.at[buf],
                             sems[buf])

        def drain_store(i, buf):
            pltpu.make_async_copy(t_hbm.at[idx_v.at[buf]], rows_v.at[buf],
                                  sems[buf]).wait()
            pltpu.sync_copy(rows_v.at[buf], out_hbm.at[pl.ds(base + i * ch,
                                                             ch)])

        start(0, 0)

        def body(ii, carry):
            i0 = ii * 2
            for b2 in range(2):
                i = i0 + b2
                nxt = i + 1

                @pl.when(nxt < n_ch)
                def _():
                    start(nxt, (b2 + 1) % 2)

                drain_store(i, b2)
            return carry

        lax.fori_loop(0, n_ch // 2, body, 0)

    return gather_k(table, idx)


def _final_body(g_ref, ap_ref, xyzT_ref, w0_ref, wgs_ref, out_ref,
                *, fb, k, tw):
    flat = g_ref[...]
    q = jnp.dot(flat[:, 128:132], wgs_ref[...],
                preferred_element_type=jnp.float32)
    r = jnp.reshape(flat, (fb, k, tw))
    gf = r[:, :, 0:128]
    gx = r[:, :, 128:131]
    ap = ap_ref[...]
    a = ap[:, None, 0:128]
    p = ap[:, None, 128:256]
    xq = xyzT_ref[0][:, None, 0:3]
    diff = xq - gx
    d = jnp.sqrt(jnp.sum(diff * diff, axis=2, keepdims=True))
    f = jnp.maximum(a + gf, 0.0)
    g = jnp.maximum(p + jnp.reshape(q, (fb, k, 128))
                    + w0_ref[...][None] * d, 0.0)
    out_ref[0] = jnp.max(f * g, axis=1)


def _final(g, ap, xyzT, w0, wgs):
    b, n, _ = xyzT.shape
    out_c = w0.shape[1]
    fb = 256
    nb = n // fb
    body = functools.partial(_final_body, fb=fb, k=KNN, tw=TW)
    return pl.pallas_call(
        body,
        grid=(b, nb),
        in_specs=[
            pl.BlockSpec((fb * KNN, TW), lambda i, j: (i * nb + j, 0)),
            pl.BlockSpec((fb, 2 * out_c), lambda i, j: (i * nb + j, 0)),
            pl.BlockSpec((1, fb, 4), lambda i, j: (i, j, 0)),
            pl.BlockSpec((1, out_c), lambda i, j: (0, 0)),
            pl.BlockSpec((4, out_c), lambda i, j: (0, 0)),
        ],
        out_specs=pl.BlockSpec((1, fb, out_c), lambda i, j: (i, j, 0)),
        out_shape=jax.ShapeDtypeStruct((b, n, out_c), jnp.float32),
        compiler_params=pltpu.CompilerParams(
            dimension_semantics=("parallel", "parallel")),
    )(g, ap, xyzT, w0, wgs)


def kernel(xyz, xyz_s, fea, fea_s, Wf, bf, Wg, bg):
    b, c, n = fea.shape
    out_c = Wf.shape[0]
    pad = jnp.zeros((b, n, 1), jnp.float32)
    xyzT = jnp.concatenate([jnp.swapaxes(xyz, 1, 2), pad], axis=2)
    xyz_sT = jnp.concatenate([jnp.swapaxes(xyz_s, 1, 2), pad], axis=2)
    wf1 = Wf[:, :c].T
    wf2 = Wf[:, c:].T
    wpad = jnp.zeros((1, out_c), jnp.float32)
    wge = jnp.concatenate([(Wg[:, 1:4] + Wg[:, 7:10]).T, wpad], axis=0)
    wgs = jnp.concatenate([(Wg[:, 4:7] - Wg[:, 7:10]).T, wpad], axis=0)
    w0 = Wg[:, 0:1].T
    bf2 = bf.reshape(1, out_c)
    bg2 = bg.reshape(1, out_c)

    table, ap = _prep(fea, fea_s, xyzT, xyz_sT, wf1, wf2, wge, bf2, bg2)
    outs = []
    for bb in range(b):
        xyzT_b = xyzT[bb:bb + 1]
        idx_b = _knn(xyzT_b, xyz_s[bb:bb + 1])
        g_b = _gather_rows(table, idx_b.reshape(-1) + bb * n)
        ap_b = ap[bb * n:(bb + 1) * n]
        outs.append(_final(g_b, ap_b, xyzT_b, w0, wgs))
    out = jnp.concatenate(outs, axis=0)
    return jnp.swapaxes(out, 1, 2)


# speedup vs baseline: 2.5235x; 1.3566x over previous
import functools

import jax
import jax.numpy as jnp
from jax import lax
from jax.experimental import pallas as pl
from jax.experimental.pallas import tpu as pltpu
from jax.experimental.pallas import tpu_sc as plsc

KNN = 16
TW = 256
NEG_INF = float("-inf")

SC_NC = 2
SC_NS = 16
SC_NW = SC_NC * SC_NS


def _prep_body(fea_ref, fea_s_ref, xyzT_ref, xyz_sT_ref, wf1_ref, wf2_ref,
               wge_ref, bf_ref, bg_ref, t_ref, ap_ref):
    fea = fea_ref[0]
    fea_s = fea_s_ref[0]
    xq = xyzT_ref[0]
    xs = xyz_sT_ref[0]
    dn = (((0,), (0,)), ((), ()))
    a = lax.dot_general(fea, wf1_ref[...], dn,
                        preferred_element_type=jnp.float32)
    ap_ref[:, 0:128] = a + bf_ref[...]
    p = jnp.dot(xq, wge_ref[...], preferred_element_type=jnp.float32)
    ap_ref[:, 128:256] = p + bg_ref[...]
    bm = lax.dot_general(fea_s, wf2_ref[...], dn,
                         preferred_element_type=jnp.float32)
    t_ref[:, 0:128] = bm
    t_ref[:, 128:132] = xs
    t_ref[:, 132:256] = jnp.zeros_like(t_ref[:, 132:256])


def _prep(fea, fea_s, xyzT, xyz_sT, wf1, wf2, wge, bf2, bg2):
    b, c, n = fea.shape
    pb = 512
    nb = n // pb
    grid = (b, nb)
    out_c = wf1.shape[1]
    return pl.pallas_call(
        _prep_body,
        grid=grid,
        in_specs=[
            pl.BlockSpec((1, c, pb), lambda i, j: (i, 0, j)),
            pl.BlockSpec((1, c, pb), lambda i, j: (i, 0, j)),
            pl.BlockSpec((1, pb, 4), lambda i, j: (i, j, 0)),
            pl.BlockSpec((1, pb, 4), lambda i, j: (i, j, 0)),
            pl.BlockSpec((c, out_c), lambda i, j: (0, 0)),
            pl.BlockSpec((c, out_c), lambda i, j: (0, 0)),
            pl.BlockSpec((4, out_c), lambda i, j: (0, 0)),
            pl.BlockSpec((1, out_c), lambda i, j: (0, 0)),
            pl.BlockSpec((1, out_c), lambda i, j: (0, 0)),
        ],
        out_specs=[
            pl.BlockSpec((pb, TW), lambda i, j: (i * nb + j, 0)),
            pl.BlockSpec((pb, 2 * out_c), lambda i, j: (i * nb + j, 0)),
        ],
        out_shape=[
            jax.ShapeDtypeStruct((b * n, TW), jnp.float32),
            jax.ShapeDtypeStruct((b * n, 2 * out_c), jnp.float32),
        ],
        compiler_params=pltpu.CompilerParams(
            dimension_semantics=("parallel", "parallel")),
    )(fea, fea_s, xyzT, xyz_sT, wf1, wf2, wge, bf2, bg2)


def _ce(va, ia, vb, ib):
    c = vb > va
    return (jnp.maximum(va, vb), jnp.where(c, ib, ia),
            jnp.minimum(va, vb), jnp.where(c, ia, ib))


def _sort16_desc(v, ix):
    k = 2
    while k <= 16:
        j = k // 2
        while j >= 1:
            for i in range(16):
                l = i ^ j
                if l > i:
                    if (i & k) == 0:
                        v[i], ix[i], v[l], ix[l] = _ce(v[i], ix[i],
                                                       v[l], ix[l])
                    else:
                        v[l], ix[l], v[i], ix[i] = _ce(v[l], ix[l],
                                                       v[i], ix[i])
            j //= 2
        k *= 2
    return v, ix


def _merge16_desc(av, ai, bv, bi):
    mv, mi = [], []
    for i in range(16):
        c = bv[15 - i] > av[i]
        mv.append(jnp.maximum(av[i], bv[15 - i]))
        mi.append(jnp.where(c, bi[15 - i], ai[i]))
    for j in (8, 4, 2, 1):
        for i in range(16):
            l = i ^ j
            if l > i:
                mv[i], mi[i], mv[l], mi[l] = _ce(mv[i], mi[i], mv[l], mi[l])
    return mv, mi


def _knn_body(xyzT_ref, xyz_s_ref, idx_ref, *, n, qb, k):
    bi = pl.program_id(0)
    xq = xyzT_ref[0]
    xs = xyz_s_ref[0]
    e = jax.lax.dot_general(
        xq[:, 0:3].astype(jnp.bfloat16), xs.astype(jnp.bfloat16),
        (((1,), (0,)), ((), ())), preferred_element_type=jnp.float32)
    inner = -2.0 * e
    a2 = (xq[:, 0:1] * xq[:, 0:1] + xq[:, 1:2] * xq[:, 1:2]) \
        + xq[:, 2:3] * xq[:, 2:3]
    b2 = (xs[0:1, :] * xs[0:1, :] + xs[1:2, :] * xs[1:2, :]) \
        + xs[2:3, :] * xs[2:3, :]
    dis = (-a2 - inner) - b2

    nw = n // 128
    lane = lax.broadcasted_iota(jnp.int32, (qb, 128), 1)
    ninf = jnp.full((qb, 128), NEG_INF, jnp.float32)
    groups = []
    for g0 in range(0, nw, 16):
        gv, gi = [], []
        for i in range(16):
            if g0 + i < nw:
                gv.append(dis[:, (g0 + i) * 128:(g0 + i + 1) * 128])
                gi.append(lane + (g0 + i) * 128)
            else:
                gv.append(ninf)
                gi.append(lane)
        groups.append(_sort16_desc(gv, gi))
    while len(groups) > 1:
        nxt = []
        for a in range(0, len(groups), 2):
            nxt.append(_merge16_desc(groups[a][0], groups[a][1],
                                     groups[a + 1][0], groups[a + 1][1]))
        groups = nxt
    wv, wi = groups[0]

    cols = []
    for t in range(k):
        m = jnp.max(wv[0], axis=1, keepdims=True)
        lsel = jnp.min(jnp.where(wv[0] == m, lane, jnp.int32(128)),
                       axis=1, keepdims=True)
        hit = lane == lsel
        cols.append(jnp.sum(jnp.where(hit, wi[0], 0), axis=1, keepdims=True))
        for j in range(15 - t):
            wv[j] = jnp.where(hit, wv[j + 1], wv[j])
            wi[j] = jnp.where(hit, wi[j + 1], wi[j])
        if t < 15:
            wv[15 - t] = jnp.where(hit, NEG_INF, wv[15 - t])
    idx_ref[0] = jnp.concatenate(cols, axis=1) + bi * n


def _knn(xyzT, xyz_s):
    b, n, _ = xyzT.shape
    qb = 128
    grid = (b, n // qb)
    body = functools.partial(_knn_body, n=n, qb=qb, k=KNN)
    return pl.pallas_call(
        body,
        grid=grid,
        in_specs=[
            pl.BlockSpec((1, qb, 4), lambda i, j: (i, j, 0)),
            pl.BlockSpec((1, 3, n), lambda i, j: (i, 0, 0)),
        ],
        out_specs=pl.BlockSpec((1, qb, KNN), lambda i, j: (i, j, 0)),
        out_shape=jax.ShapeDtypeStruct((b, n, KNN), jnp.int32),
        compiler_params=pltpu.CompilerParams(
            dimension_semantics=("parallel", "parallel")),
    )(xyzT, xyz_s)


def _gather_rows(table, idx):
    m = idx.shape[0]
    tw = table.shape[1]
    per_w = m // SC_NW
    ch = 128
    n_ch = per_w // ch
    mesh = plsc.VectorSubcoreMesh(core_axis_name="c", subcore_axis_name="s")

    @functools.partial(
        pl.kernel,
        out_type=jax.ShapeDtypeStruct((m, tw), jnp.float32),
        mesh=mesh,
        scratch_types=[
            pltpu.VMEM((2, ch), jnp.int32),
            pltpu.VMEM((2, ch, tw), jnp.float32),
            pltpu.SemaphoreType.DMA,
            pltpu.SemaphoreType.DMA,
        ],
    )
    def gather_k(t_hbm, idx_hbm, out_hbm, idx_v, rows_v, sem0, sem1):
        wid = lax.axis_index("s") * SC_NC + lax.axis_index("c")
        base = wid * per_w
        sems = (sem0, sem1)

        def start(i, buf):
            pltpu.sync_copy(idx_hbm.at[pl.ds(base + i * ch, ch)],
                            idx_v.at[buf])
            pltpu.async_copy(t_hbm.at[idx_v.at[buf]], rows_v.at[buf],
                             sems[buf])

        def drain_store(i, buf):
            pltpu.make_async_copy(t_hbm.at[idx_v.at[buf]], rows_v.at[buf],
                                  sems[buf]).wait()
            pltpu.sync_copy(rows_v.at[buf], out_hbm.at[pl.ds(base + i * ch,
                                                             ch)])

        start(0, 0)

        def body(ii, carry):
            i0 = ii * 2
            for b2 in range(2):
                i = i0 + b2
                nxt = i + 1

                @pl.when(nxt < n_ch)
                def _():
                    start(nxt, (b2 + 1) % 2)

                drain_store(i, b2)
            return carry

        lax.fori_loop(0, n_ch // 2, body, 0)

    return gather_k(table, idx)


def _final_body(g_ref, ap_ref, xyzT_ref, w0_ref, wgs_ref, out_ref,
                *, fb, k, tw):
    flat = g_ref[...]
    q = jnp.dot(flat[:, 128:132], wgs_ref[...],
                preferred_element_type=jnp.float32)
    r = jnp.reshape(flat, (fb, k, tw))
    gf = r[:, :, 0:128]
    gx = r[:, :, 128:131]
    ap = ap_ref[...]
    a = ap[:, None, 0:128]
    p = ap[:, None, 128:256]
    xq = xyzT_ref[0][:, None, 0:3]
    diff = xq - gx
    d = jnp.sqrt(jnp.sum(diff * diff, axis=2, keepdims=True))
    f = jnp.maximum(a + gf, 0.0)
    g = jnp.maximum(p + jnp.reshape(q, (fb, k, 128))
                    + w0_ref[...][None] * d, 0.0)
    out_ref[0] = jnp.max(f * g, axis=1)


def _final(g, ap, xyzT, w0, wgs):
    b, n, _ = xyzT.shape
    out_c = w0.shape[1]
    fb = 256
    nb = n // fb
    body = functools.partial(_final_body, fb=fb, k=KNN, tw=TW)
    return pl.pallas_call(
        body,
        grid=(b, nb),
        in_specs=[
            pl.BlockSpec((fb * KNN, TW), lambda i, j: (i * nb + j, 0)),
            pl.BlockSpec((fb, 2 * out_c), lambda i, j: (i * nb + j, 0)),
            pl.BlockSpec((1, fb, 4), lambda i, j: (i, j, 0)),
            pl.BlockSpec((1, out_c), lambda i, j: (0, 0)),
            pl.BlockSpec((4, out_c), lambda i, j: (0, 0)),
        ],
        out_specs=pl.BlockSpec((1, fb, out_c), lambda i, j: (i, j, 0)),
        out_shape=jax.ShapeDtypeStruct((b, n, out_c), jnp.float32),
        compiler_params=pltpu.CompilerParams(
            dimension_semantics=("parallel", "parallel")),
    )(g, ap, xyzT, w0, wgs)


def kernel(xyz, xyz_s, fea, fea_s, Wf, bf, Wg, bg):
    b, c, n = fea.shape
    out_c = Wf.shape[0]
    pad = jnp.zeros((b, n, 1), jnp.float32)
    xyzT = jnp.concatenate([jnp.swapaxes(xyz, 1, 2), pad], axis=2)
    xyz_sT = jnp.concatenate([jnp.swapaxes(xyz_s, 1, 2), pad], axis=2)
    wf1 = Wf[:, :c].T
    wf2 = Wf[:, c:].T
    wpad = jnp.zeros((1, out_c), jnp.float32)
    wge = jnp.concatenate([(Wg[:, 1:4] + Wg[:, 7:10]).T, wpad], axis=0)
    wgs = jnp.concatenate([(Wg[:, 4:7] - Wg[:, 7:10]).T, wpad], axis=0)
    w0 = Wg[:, 0:1].T
    bf2 = bf.reshape(1, out_c)
    bg2 = bg.reshape(1, out_c)

    table, ap = _prep(fea, fea_s, xyzT, xyz_sT, wf1, wf2, wge, bf2, bg2)
    nh = 2
    hn = n // nh
    outs = []
    for bb in range(b):
        for hh in range(nh):
            xyzT_c = xyzT[bb:bb + 1, hh * hn:(hh + 1) * hn]
            idx_c = _knn(xyzT_c, xyz_s[bb:bb + 1])
            g_c = _gather_rows(table, idx_c.reshape(-1) + bb * n)
            ap_c = ap[bb * n + hh * hn:bb * n + (hh + 1) * hn]
            outs.append(_final(g_c, ap_c, xyzT_c, w0, wgs))
    out = jnp.concatenate(
        [jnp.concatenate(outs[bb * nh:(bb + 1) * nh], axis=1)
         for bb in range(b)], axis=0)
    return jnp.swapaxes(out, 1, 2)
